# baseline (speedup 1.0000x reference)
import functools

import jax
import jax.numpy as jnp
from jax import lax
from jax.experimental import pallas as pl
from jax.experimental.pallas import tpu as pltpu
from jax.experimental.pallas import tpu_sc as plsc

N_V = 100000
N_P = 400000
N_DOWN = 25000
C = 128
H = 64

RB = 2000
GRID = N_V // RB
RB25 = 1000
NC = 2
NS = 16
PROW = 128
NPR = N_P // PROW
VROW = 80
NVR = N_V // VROW
ZR = 64


def _leaky(x):
    return jnp.where(x > 0, x, 0.1 * x)


def _stats_of(y):
    s = jnp.sum(y, axis=0, keepdims=True)
    ss = jnp.sum(y * y, axis=0, keepdims=True)
    z = jnp.zeros((6, y.shape[1]), jnp.float32)
    return jnp.concatenate([s, ss, z], axis=0)


def _acc_stats(st_ref, y):
    i = pl.program_id(0)

    @pl.when(i == 0)
    def _():
        st_ref[...] = jnp.zeros_like(st_ref)

    st_ref[...] += _stats_of(y)


def _k1_body(x_ref, w_ref, y_ref, st_ref):
    y = jnp.dot(x_ref[...], w_ref[...], preferred_element_type=jnp.float32)
    y_ref[...] = y
    _acc_stats(st_ref, y)


def _k2_body(y1_ref, cs_ref, w_ref, y2_ref, st_ref):
    sc = cs_ref[0:1, :]
    sh = cs_ref[1:2, :]
    a = jax.nn.relu(y1_ref[...] * sc + sh)
    y2 = jnp.dot(a, w_ref[...], preferred_element_type=jnp.float32)
    y2_ref[...] = y2
    _acc_stats(st_ref, y2)


def _k3_body(y2_ref, x_ref, cs_ref, w_ref, v1_ref, y3_ref, st_ref):
    sc = cs_ref[0:1, :]
    sh = cs_ref[1:2, :]
    v1 = jax.nn.relu(y2_ref[...] * sc + sh + x_ref[...])
    v1_ref[...] = v1
    y3 = jnp.dot(v1, w_ref[...], preferred_element_type=jnp.float32)
    y3_ref[...] = y3
    _acc_stats(st_ref, y3)


def _k5_body(y4_ref, v1_ref, x_ref, cs_ref, wp1_ref, bp1_ref,
             v_ref, vsc_ref, g1_ref):
    sc = cs_ref[0:1, :]
    sh = cs_ref[1:2, :]
    vv = jax.nn.relu(y4_ref[...] * sc + sh + v1_ref[...])
    v_ref[...] = vv
    vsc_ref[0] = vv[:, :H]
    vsc_ref[1] = vv[:, H:]
    f = x_ref[...] + vv
    g1_ref[...] = (jnp.dot(f, wp1_ref[...], preferred_element_type=jnp.float32)
                   + bp1_ref[0:1, :])


def _t3a_body(ds_ref, ic_ref, st_ref):
    h1 = _leaky(ds_ref[...] * ic_ref[...])
    _acc_stats(st_ref, h1)


def _t3b_body(ds_ref, ic_ref, cs1_ref, w_ref, b_ref, st_ref):
    h1 = _leaky(ds_ref[...] * ic_ref[...])
    h1n = h1 * cs1_ref[0:1, :] + cs1_ref[1:2, :]
    h2p = _leaky(jnp.dot(h1n, w_ref[...], preferred_element_type=jnp.float32)
                 + b_ref[0:1, :])
    _acc_stats(st_ref, h2p)


def _t3c_body(ds_ref, ic_ref, cs1_ref, w_ref, b_ref, cs2_ref, h2_ref):
    h1 = _leaky(ds_ref[...] * ic_ref[...])
    h1n = h1 * cs1_ref[0:1, :] + cs1_ref[1:2, :]
    h2p = _leaky(jnp.dot(h1n, w_ref[...], preferred_element_type=jnp.float32)
                 + b_ref[0:1, :])
    h2_ref[...] = h2p * cs2_ref[0:1, :] + cs2_ref[1:2, :]


def _tc4_body(x_ref, vsc_ref, h2g_ref, wli_ref, wp3_ref, wo1a_ref, wo1b_ref,
              wo2_ref, bv_ref, losc_ref):
    v = jnp.concatenate([vsc_ref[0], vsc_ref[1]], axis=1)
    f = x_ref[...] + v
    idt = _leaky(jnp.dot(f, wli_ref[...], preferred_element_type=jnp.float32)
                 + bv_ref[0:1, :])
    h3 = _leaky(jnp.dot(h2g_ref[...], wp3_ref[...],
                        preferred_element_type=jnp.float32) + bv_ref[1:2, :])
    t = _leaky(jnp.dot(idt, wo1a_ref[...], preferred_element_type=jnp.float32)
               + jnp.dot(h3, wo1b_ref[...], preferred_element_type=jnp.float32)
               + bv_ref[2:3, :])
    lo = (jnp.dot(t, wo2_ref[...], preferred_element_type=jnp.float32)
          + bv_ref[3:4, :])
    losc_ref[0] = lo[:, :H]
    losc_ref[1] = lo[:, H:]


def _tc5_body(sv_ref, sl_ref, ic_ref, nf_ref, pf_ref):
    ic = ic_ref[...]
    pf = jnp.concatenate([sl_ref[0], sl_ref[1]], axis=1) * ic
    vf = jnp.concatenate([sv_ref[0], sv_ref[1]], axis=1) * ic
    pf_ref[...] = pf
    nf_ref[...] = pf + vf


def _bspec(shape, idx):
    return pl.BlockSpec(shape, idx)


_FULLW = pl.BlockSpec((C, C), lambda i: (0, 0))
_FULLH = pl.BlockSpec((H, H), lambda i: (0, 0))
_ROWB = pl.BlockSpec((RB, C), lambda i: (i, 0))
_ROWBH = pl.BlockSpec((RB, H), lambda i: (i, 0))
_CS = pl.BlockSpec((8, C), lambda i: (0, 0))
_CSH = pl.BlockSpec((8, H), lambda i: (0, 0))
_ST = pl.BlockSpec((8, C), lambda i: (0, 0))
_STH = pl.BlockSpec((8, H), lambda i: (0, 0))
_SPLITB = pl.BlockSpec((2, RB, H), lambda i: (0, i, 0))
_IC = pl.BlockSpec((RB, 1), lambda i: (i, 0))


def _call(body, in_specs, out_specs, out_shapes, grid=GRID):
    return pl.pallas_call(
        body,
        grid=(grid,),
        in_specs=in_specs,
        out_specs=out_specs,
        out_shape=out_shapes,
    )


def _bn_coeffs(st, n, g, b):
    s, ss = st[0], st[1]
    m = s / n
    var = ss / n - m * m
    sc = g / jnp.sqrt(var + 1e-5)
    sh = b - m * sc
    return jnp.stack([sc, sh] + [jnp.zeros_like(sc)] * 6, axis=0)


def _share(s, n, parts):
    base = n // parts
    rem = n % parts
    cnt = base + jnp.where(s < rem, 1, 0)
    start = s * base + jnp.minimum(s, rem)
    return start, cnt


def _zero_vmem(ref, rows, width):
    for r in range(rows):
        for j in range(width // 16):
            ref[r, pl.ds(j * 16, 16)] = jnp.zeros((16,), jnp.float32)


def _fill_ones(ref, rows, width):
    for r in range(rows):
        for j in range(width // 16):
            ref[r, pl.ds(j * 16, 16)] = jnp.ones((16,), jnp.float32)


def _sc_mesh():
    return plsc.VectorSubcoreMesh(core_axis_name="c", subcore_axis_name="s",
                                  num_cores=NC, num_subcores=NS)


def _zero_spmem_rows(zbuf, shref, n_rows, s, zr=ZR):
    nch = n_rows // zr
    z0, zc = _share(s, nch, NS)

    def zbody(i, _):
        pltpu.sync_copy(zbuf, shref.at[pl.ds(i * zr, zr)])
        return 0

    lax.fori_loop(z0, z0 + zc, zbody, 0)
    rem = n_rows - nch * zr
    if rem:
        @pl.when(s == 0)
        def _():
            pltpu.sync_copy(zbuf.at[pl.ds(0, rem)],
                            shref.at[pl.ds(nch * zr, rem)])


def _copy_spmem_rows(shref, dst, dst_base, n_rows, s):
    nch = n_rows // ZR
    w0, wc = _share(s, nch, NS)

    def wbody(i, _):
        pltpu.sync_copy(shref.at[pl.ds(i * ZR, ZR)],
                        dst.at[pl.ds(dst_base + i * ZR, ZR)])
        return 0

    lax.fori_loop(w0, w0 + wc, wbody, 0)
    rem = n_rows - nch * ZR
    if rem:
        @pl.when(s == 0)
        def _():
            pltpu.sync_copy(shref.at[pl.ds(nch * ZR, rem)],
                            dst.at[pl.ds(dst_base + nch * ZR, rem)])


def _make_scatter_stage():
    out_type = [jax.ShapeDtypeStruct((2 * N_DOWN, H), jnp.float32)]

    SROW = 80
    NSR = N_P // SROW

    def body(table, last1, scale1, sums,
             il0, il1, il2, is0, is1, is2, r0, r1, r2,
             sil0, sil1, sil2, sis0, sis1, sis2,
             sg0, sg1, sg2, ss0, ss1, ss2, acc):
        c = lax.axis_index("c")
        s = lax.axis_index("s")
        off = c * N_V
        ils = [il0, il1, il2]
        iss = [is0, is1, is2]
        rs = [r0, r1, r2]
        sils = [sil0, sil1, sil2]
        siss = [sis0, sis1, sis2]
        sgs = [sg0, sg1, sg2]
        sss = [ss0, ss1, ss2]

        _zero_vmem(r0, SROW, H)
        _zero_spmem_rows(r0, acc, N_DOWN, s, SROW)
        plsc.subcore_barrier()

        g0, gc = _share(s, NSR, NS)

        def start_idx(g, b):
            pltpu.async_copy(last1.at[pl.ds((g0 + g) * SROW, SROW)],
                             ils[b], sils[b])
            pltpu.async_copy(scale1.at[pl.ds((g0 + g) * SROW, SROW)],
                             iss[b], siss[b])

        def wait_idx(g, b):
            pltpu.make_async_copy(last1.at[pl.ds((g0 + g) * SROW, SROW)],
                                  ils[b], sils[b]).wait()
            pltpu.make_async_copy(scale1.at[pl.ds((g0 + g) * SROW, SROW)],
                                  iss[b], siss[b]).wait()

        def wait_scatter(b):
            pltpu.make_async_copy(rs[b], acc.at[iss[b]], sss[b]).wait()

        @pl.when(gc > 0)
        def _():
            start_idx(0, 0)

        def outer(o, _):
            for b in range(3):
                g = o * 3 + b

                @pl.when(g < gc)
                def _(g=g, b=b):
                    wait_idx(g, b)
                    for j in range(SROW // 16):
                        ils[b][pl.ds(j * 16, 16)] = (
                            ils[b][pl.ds(j * 16, 16)] + off)

                    @pl.when(g > 1)
                    def _():
                        wait_scatter((b + 1) % 3)

                    pltpu.async_copy(table.at[ils[b]], rs[b], sgs[b])

                    @pl.when(g > 0)
                    def _():
                        bp = (b + 2) % 3
                        pltpu.make_async_copy(table.at[ils[bp]], rs[bp],
                                              sgs[bp]).wait()
                        pltpu.async_copy(rs[bp], acc.at[iss[bp]], sss[bp],
                                         add=True)

                    @pl.when(g + 1 < gc)
                    def _():
                        start_idx(g + 1, (b + 1) % 3)
            return 0

        lax.fori_loop(0, (gc + 2) // 3, outer, 0)
        for b in range(3):
            @pl.when((gc > 1) & (lax.rem(gc - 2, 3) == b))
            def _(b=b):
                wait_scatter(b)

            @pl.when((gc > 0) & (lax.rem(gc - 1, 3) == b))
            def _(b=b):
                pltpu.make_async_copy(table.at[ils[b]], rs[b], sgs[b]).wait()
                pltpu.sync_copy(rs[b], acc.at[iss[b]], add=True)

        plsc.subcore_barrier()
        _copy_spmem_rows(acc, sums, c * N_DOWN, N_DOWN, s)

    scratch = (
        [pltpu.VMEM((SROW,), jnp.int32) for _ in range(6)]
        + [pltpu.VMEM((SROW, H), jnp.float32) for _ in range(3)]
        + [pltpu.SemaphoreType.DMA for _ in range(12)]
        + [pltpu.VMEM_SHARED((N_DOWN, H), jnp.float32)]
    )
    return pl.kernel(body, out_type=out_type, mesh=_sc_mesh(),
                     scratch_types=scratch,
                     compiler_params=pltpu.CompilerParams(
                         use_tc_tiling_on_sc=False))


def _make_counts_stage():
    HALF = N_V // 2
    out_type = [jax.ShapeDtypeStruct((N_V, 16), jnp.float32),
                jax.ShapeDtypeStruct((N_DOWN, 16), jnp.float32)]

    def body(inv1, scale1, cnt100, cnt25, i80a, i80b, ia, ib, s0, s1,
             ones80, ones, zbuf, acc):
        c = lax.axis_index("c")
        s = lax.axis_index("s")
        _fill_ones(ones, PROW, 16)
        _fill_ones(ones80, VROW, 16)
        _zero_vmem(zbuf, ZR, 16)
        sms = [s0, s1]

        @pl.when(c == 0)
        def _():
            idxs = [i80a, i80b]
            for kk in range(2):
                base = kk * HALF
                _zero_spmem_rows(zbuf, acc, HALF + 8, s)
                plsc.subcore_barrier()
                g0, gc = _share(s, NVR, NS)

                def start_ld(g, b):
                    pltpu.async_copy(inv1.at[pl.ds((g0 + g) * VROW, VROW)],
                                     idxs[b], sms[b])

                def wait_ld(g, b):
                    pltpu.make_async_copy(
                        inv1.at[pl.ds((g0 + g) * VROW, VROW)],
                        idxs[b], sms[b]).wait()

                @pl.when(gc > 0)
                def _():
                    start_ld(0, 0)

                def outer(o, _):
                    for b in range(2):
                        g = o * 2 + b

                        @pl.when(g < gc)
                        def _(g=g, b=b):
                            wait_ld(g, b)

                            @pl.when(g + 1 < gc)
                            def _():
                                start_ld(g + 1, 1 - b)

                            for j in range(VROW // 16):
                                t = idxs[b][pl.ds(j * 16, 16)] - base
                                ok = (t >= 0) & (t < HALF)
                                idxs[b][pl.ds(j * 16, 16)] = jnp.where(
                                    ok, t, HALF)
                            pltpu.sync_copy(ones80, acc.at[idxs[b]],
                                            add=True)
                    return 0

                lax.fori_loop(0, (gc + 1) // 2, outer, 0)
                plsc.subcore_barrier()
                _copy_spmem_rows(acc, cnt100, base, HALF, s)
                plsc.subcore_barrier()

        @pl.when(c == 1)
        def _():
            idxs = [ia, ib]
            _zero_spmem_rows(zbuf, acc, N_DOWN, s)
            plsc.subcore_barrier()
            g0, gc = _share(s, NPR, NS)

            def start_ld(g, b):
                pltpu.async_copy(scale1.at[pl.ds((g0 + g) * PROW, PROW)],
                                 idxs[b], sms[b])

            def wait_ld(g, b):
                pltpu.make_async_copy(scale1.at[pl.ds((g0 + g) * PROW, PROW)],
                                      idxs[b], sms[b]).wait()

            @pl.when(gc > 0)
            def _():
                start_ld(0, 0)

            def outer(o, _):
                for b in range(2):
                    g = o * 2 + b

                    @pl.when(g < gc)
                    def _(g=g, b=b):
                        wait_ld(g, b)

                        @pl.when(g + 1 < gc)
                        def _():
                            start_ld(g + 1, 1 - b)

                        pltpu.sync_copy(ones, acc.at[idxs[b]], add=True)
                return 0

            lax.fori_loop(0, (gc + 1) // 2, outer, 0)
            plsc.subcore_barrier()
            _copy_spmem_rows(acc, cnt25, 0, N_DOWN, s)

    scratch = [
        pltpu.VMEM((VROW,), jnp.int32),
        pltpu.VMEM((VROW,), jnp.int32),
        pltpu.VMEM((PROW,), jnp.int32),
        pltpu.VMEM((PROW,), jnp.int32),
        pltpu.SemaphoreType.DMA,
        pltpu.SemaphoreType.DMA,
        pltpu.VMEM((VROW, 16), jnp.float32),
        pltpu.VMEM((PROW, 16), jnp.float32),
        pltpu.VMEM((ZR, 16), jnp.float32),
        pltpu.VMEM_SHARED((HALF + 8, 16), jnp.float32),
    ]
    return pl.kernel(body, out_type=out_type, mesh=_sc_mesh(),
                     scratch_types=scratch,
                     compiler_params=pltpu.CompilerParams(
                         use_tc_tiling_on_sc=False))


def _make_down_stage():
    out_type = [jax.ShapeDtypeStruct((N_V, H), jnp.float32)]

    def body(g1, inv1, dsum, i0, i1, i2, v0, v1, v2,
             si0, si1, si2, sv0, sv1, sv2, ss0, ss1, ss2, zbuf, acc):
        c = lax.axis_index("c")
        s = lax.axis_index("s")
        _zero_vmem(zbuf, ZR, H)
        idxs = [i0, i1, i2]
        vs = [v0, v1, v2]
        sis = [si0, si1, si2]
        svs = [sv0, sv1, sv2]
        sss = [ss0, ss1, ss2]

        for kk in range(2):
            base = (2 * c + kk) * N_DOWN
            _zero_spmem_rows(zbuf, acc, N_DOWN + 8, s)
            plsc.subcore_barrier()

            g0, gc = _share(s, NVR, NS)

            def start_ld(g, b):
                pltpu.async_copy(inv1.at[pl.ds((g0 + g) * VROW, VROW)],
                                 idxs[b], sis[b])
                pltpu.async_copy(g1.at[pl.ds((g0 + g) * VROW, VROW)],
                                 vs[b], svs[b])

            def wait_ld(g, b):
                pltpu.make_async_copy(inv1.at[pl.ds((g0 + g) * VROW, VROW)],
                                      idxs[b], sis[b]).wait()
                pltpu.make_async_copy(g1.at[pl.ds((g0 + g) * VROW, VROW)],
                                      vs[b], svs[b]).wait()

            def wait_scatter(b):
                pltpu.make_async_copy(vs[b], acc.at[idxs[b]], sss[b]).wait()

            @pl.when(gc > 0)
            def _():
                start_ld(0, 0)

            def outer(o, _):
                for b in range(3):
                    g = o * 3 + b

                    @pl.when(g < gc)
                    def _(g=g, b=b):
                        wait_ld(g, b)
                        for j in range(VROW // 16):
                            t = idxs[b][pl.ds(j * 16, 16)] - base
                            ok = (t >= 0) & (t < N_DOWN)
                            idxs[b][pl.ds(j * 16, 16)] = jnp.where(
                                ok, t, N_DOWN)
                        pltpu.async_copy(vs[b], acc.at[idxs[b]], sss[b],
                                         add=True)

                        @pl.when(g > 1)
                        def _():
                            wait_scatter((b + 1) % 3)

                        @pl.when(g + 1 < gc)
                        def _():
                            start_ld(g + 1, (b + 1) % 3)
                return 0

            lax.fori_loop(0, (gc + 2) // 3, outer, 0)
            for b in range(3):
                @pl.when((gc > 1) & (lax.rem(gc - 2, 3) == b))
                def _(b=b):
                    wait_scatter(b)

                @pl.when((gc > 0) & (lax.rem(gc - 1, 3) == b))
                def _(b=b):
                    wait_scatter(b)

            plsc.subcore_barrier()

            _copy_spmem_rows(acc, dsum, base, N_DOWN, s)
            plsc.subcore_barrier()

    scratch = (
        [pltpu.VMEM((VROW,), jnp.int32) for _ in range(3)]
        + [pltpu.VMEM((VROW, H), jnp.float32) for _ in range(3)]
        + [pltpu.SemaphoreType.DMA for _ in range(9)]
        + [pltpu.VMEM((ZR, H), jnp.float32),
           pltpu.VMEM_SHARED((N_DOWN + 8, H), jnp.float32)]
    )
    return pl.kernel(body, out_type=out_type, mesh=_sc_mesh(),
                     scratch_types=scratch,
                     compiler_params=pltpu.CompilerParams(
                         use_tc_tiling_on_sc=False))


def _make_gather_stage(width, n_chunks, per_row):
    out_type = [jax.ShapeDtypeStruct((n_chunks * per_row, width),
                                     jnp.float32)]

    def body(tab, idx1, out, i0, i1, b0, b1, si0, si1, sg0, sg1, sw0, sw1):
        c = lax.axis_index("c")
        s = lax.axis_index("s")
        wid = s * NC + c
        g0, gc = _share(wid, n_chunks, NC * NS)
        idxs = [i0, i1]
        bufs = [b0, b1]
        sis = [si0, si1]
        sgs = [sg0, sg1]
        sws = [sw0, sw1]

        def start_idx(g, b):
            pltpu.async_copy(idx1.at[pl.ds((g0 + g) * per_row, per_row)],
                             idxs[b], sis[b])

        def wait_idx(g, b):
            pltpu.make_async_copy(idx1.at[pl.ds((g0 + g) * per_row, per_row)],
                                  idxs[b], sis[b]).wait()

        def start_write(g, b):
            pltpu.async_copy(bufs[b],
                             out.at[pl.ds((g0 + g) * per_row, per_row)],
                             sws[b])

        def wait_write(g, b):
            pltpu.make_async_copy(
                bufs[b], out.at[pl.ds((g0 + g) * per_row, per_row)],
                sws[b]).wait()

        @pl.when(gc > 0)
        def _():
            start_idx(0, 0)

        def outer(o, _):
            for b in range(2):
                g = o * 2 + b

                @pl.when(g < gc)
                def _(g=g, b=b):
                    wait_idx(g, b)

                    @pl.when(g > 1)
                    def _():
                        wait_write(g - 2, b)

                    pltpu.async_copy(tab.at[idxs[b]], bufs[b], sgs[b])

                    @pl.when(g > 0)
                    def _():
                        pltpu.make_async_copy(tab.at[idxs[1 - b]],
                                              bufs[1 - b], sgs[1 - b]).wait()
                        start_write(g - 1, 1 - b)

                    @pl.when(g + 1 < gc)
                    def _():
                        start_idx(g + 1, 1 - b)
            return 0

        lax.fori_loop(0, (gc + 1) // 2, outer, 0)
        for b in range(2):
            @pl.when((gc > 1) & (lax.rem(gc - 2, 2) == b))
            def _(b=b):
                wait_write(gc - 2, b)

            @pl.when((gc > 0) & (lax.rem(gc - 1, 2) == b))
            def _(b=b):
                pltpu.make_async_copy(tab.at[idxs[b]], bufs[b],
                                      sgs[b]).wait()
                pltpu.sync_copy(bufs[b],
                                out.at[pl.ds((g0 + gc - 1) * per_row,
                                             per_row)])

    scratch = [
        pltpu.VMEM((per_row,), jnp.int32),
        pltpu.VMEM((per_row,), jnp.int32),
        pltpu.VMEM((per_row, width), jnp.float32),
        pltpu.VMEM((per_row, width), jnp.float32),
        pltpu.SemaphoreType.DMA,
        pltpu.SemaphoreType.DMA,
        pltpu.SemaphoreType.DMA,
        pltpu.SemaphoreType.DMA,
        pltpu.SemaphoreType.DMA,
        pltpu.SemaphoreType.DMA,
    ]
    return pl.kernel(body, out_type=out_type, mesh=_sc_mesh(),
                     scratch_types=scratch,
                     compiler_params=pltpu.CompilerParams(
                         use_tc_tiling_on_sc=False))


def kernel(features, coors, coors_inv_last, coors_inv_scale, params):
    p = params
    nvf = jnp.float32(N_V)

    y1, st1 = _call(_k1_body, [_ROWB, _FULLW], [_ROWB, _ST],
                    [jax.ShapeDtypeStruct((N_V, C), jnp.float32),
                     jax.ShapeDtypeStruct((8, C), jnp.float32)])(
                         features, p['W1a'])
    cs1 = _bn_coeffs(st1, nvf, p['g1a'], p['b1a'])
    y2, st2 = _call(_k2_body, [_ROWB, _CS, _FULLW], [_ROWB, _ST],
                    [jax.ShapeDtypeStruct((N_V, C), jnp.float32),
                     jax.ShapeDtypeStruct((8, C), jnp.float32)])(
                         y1, cs1, p['W1b'])
    cs2 = _bn_coeffs(st2, nvf, p['g1b'], p['b1b'])
    v1, y3, st3 = _call(_k3_body, [_ROWB, _ROWB, _CS, _FULLW],
                        [_ROWB, _ROWB, _ST],
                        [jax.ShapeDtypeStruct((N_V, C), jnp.float32),
                         jax.ShapeDtypeStruct((N_V, C), jnp.float32),
                         jax.ShapeDtypeStruct((8, C), jnp.float32)])(
                            y2, features, cs2, p['W2a'])
    cs3 = _bn_coeffs(st3, nvf, p['g2a'], p['b2a'])
    y4, st4 = _call(_k2_body, [_ROWB, _CS, _FULLW], [_ROWB, _ST],
                    [jax.ShapeDtypeStruct((N_V, C), jnp.float32),
                     jax.ShapeDtypeStruct((8, C), jnp.float32)])(
                         y3, cs3, p['W2b'])
    cs4 = _bn_coeffs(st4, nvf, p['g2b'], p['b2b'])
    bp1v = jnp.stack([p['bp1']] + [jnp.zeros((H,), jnp.float32)] * 7, axis=0)
    wp1spec = pl.BlockSpec((C, H), lambda i: (0, 0))
    v, vsc, g1 = _call(_k5_body,
                       [_ROWB, _ROWB, _ROWB, _CS, wp1spec, _CSH],
                       [_ROWB, _SPLITB, _ROWBH],
                       [jax.ShapeDtypeStruct((N_V, C), jnp.float32),
                        jax.ShapeDtypeStruct((2, N_V, H), jnp.float32),
                        jax.ShapeDtypeStruct((N_V, H), jnp.float32)])(
                           y4, v1, features, cs4, p['Wp1'], bp1v)

    co = coors.astype(jnp.int32)
    b = co[:, 0]
    xyz = co[:, 1:] // 2
    keyv = ((b * 64 + xyz[:, 0]) * 64 + xyz[:, 1]) * 64 + xyz[:, 2]
    order = jnp.argsort(keyv)
    sk = keyv[order]
    flag = jnp.concatenate(
        [jnp.zeros((1,), sk.dtype), (sk[1:] != sk[:-1]).astype(sk.dtype)])
    gid = jnp.cumsum(flag)
    inv = jnp.zeros((N_V,), jnp.int32).at[order].set(gid.astype(jnp.int32))
    n_u = (gid[-1] + 1).astype(jnp.int32)

    vtab = vsc.reshape(2 * N_V, H)

    sumv, = _make_scatter_stage()(vtab, coors_inv_last, coors_inv_scale)
    cnt100v, cnt25v = _make_counts_stage()(inv, coors_inv_scale)
    invc25 = (1.0 / jnp.clip(cnt25v[:, 0], 1.0)).reshape(N_DOWN, 1)

    dsum, = _make_down_stage()(g1, inv)
    invc100 = (1.0 / jnp.clip(cnt100v[:, 0], 1.0)).reshape(N_V, 1)

    nf = n_u.astype(jnp.float32)
    st_h1, = _call(_t3a_body, [_ROWBH, _IC], [_STH],
                   [jax.ShapeDtypeStruct((8, H), jnp.float32)])(dsum, invc100)
    csA = _bn_coeffs(st_h1, nf, p['g1'], p['be1'])
    bp2v = jnp.stack([p['bp2']] + [jnp.zeros((H,), jnp.float32)] * 7, axis=0)
    st_h2, = _call(_t3b_body, [_ROWBH, _IC, _CSH, _FULLH, _CSH], [_STH],
                   [jax.ShapeDtypeStruct((8, H), jnp.float32)])(
                       dsum, invc100, csA, p['Wp2'], bp2v)
    c2 = _leaky(csA[1] @ p['Wp2'] + p['bp2'])
    ng = nvf - nf
    s2m = st_h2[0] - ng * c2
    ss2m = st_h2[1] - ng * c2 * c2
    csB = _bn_coeffs(jnp.stack([s2m, ss2m] + [jnp.zeros((H,))] * 6), nf,
                     p['g2'], p['be2'])
    h2, = _call(_t3c_body, [_ROWBH, _IC, _CSH, _FULLH, _CSH, _CSH], [_ROWBH],
                [jax.ShapeDtypeStruct((N_V, H), jnp.float32)])(
                    dsum, invc100, csA, p['Wp2'], bp2v, csB)

    h2g, = _make_gather_stage(H, NVR, VROW)(h2, inv)

    bv = jnp.stack([p['bli'], p['bp3'], p['bo1'], p['bo2']]
                   + [jnp.zeros((C,), jnp.float32)] * 4, axis=0)
    wp3spec = pl.BlockSpec((H, C), lambda i: (0, 0))
    losc, = _call(_tc4_body,
                  [_ROWB, _SPLITB, _ROWBH, _FULLW, wp3spec, _FULLW, _FULLW,
                   _FULLW, _CS],
                  [_SPLITB],
                  [jax.ShapeDtypeStruct((2, N_V, H), jnp.float32)])(
                      features, vsc, h2g, p['Wli'], p['Wp3'],
                      p['Wo1'][:C], p['Wo1'][C:], p['Wo2'], bv)

    sumlo, = _make_scatter_stage()(losc.reshape(2 * N_V, H),
                                   coors_inv_last, coors_inv_scale)

    svr = sumv.reshape(2, N_DOWN, H)
    slr = sumlo.reshape(2, N_DOWN, H)
    spl25 = pl.BlockSpec((2, RB25, H), lambda i: (0, i, 0))
    ic25 = pl.BlockSpec((RB25, 1), lambda i: (i, 0))
    row25 = pl.BlockSpec((RB25, C), lambda i: (i, 0))
    new_feats, p_fea = _call(
        _tc5_body, [spl25, spl25, ic25], [row25, row25],
        [jax.ShapeDtypeStruct((N_DOWN, C), jnp.float32),
         jax.ShapeDtypeStruct((N_DOWN, C), jnp.float32)],
        grid=N_DOWN // RB25)(svr, slr, invc25)

    pts_feat_f, = _make_gather_stage(C, NPR, PROW)(p_fea, coors_inv_scale)

    return (new_feats, v, pts_feat_f)


# speedup vs baseline: 1.0251x; 1.0251x over previous
import functools

import jax
import jax.numpy as jnp
from jax import lax
from jax.experimental import pallas as pl
from jax.experimental.pallas import tpu as pltpu
from jax.experimental.pallas import tpu_sc as plsc

N_V = 100000
N_P = 400000
N_DOWN = 25000
C = 128
H = 64

RB = 2000
GRID = N_V // RB
RB25 = 1000
NC = 2
NS = 16
PROW = 128
NPR = N_P // PROW
VROW = 80
NVR = N_V // VROW
ZR = 64


def _leaky(x):
    return jnp.where(x > 0, x, 0.1 * x)


def _stats_of(y):
    s = jnp.sum(y, axis=0, keepdims=True)
    ss = jnp.sum(y * y, axis=0, keepdims=True)
    z = jnp.zeros((6, y.shape[1]), jnp.float32)
    return jnp.concatenate([s, ss, z], axis=0)


def _acc_stats(st_ref, y):
    i = pl.program_id(0)

    @pl.when(i == 0)
    def _():
        st_ref[...] = jnp.zeros_like(st_ref)

    st_ref[...] += _stats_of(y)


def _k1_body(x_ref, w_ref, y_ref, st_ref):
    y = jnp.dot(x_ref[...], w_ref[...], preferred_element_type=jnp.float32)
    y_ref[...] = y
    _acc_stats(st_ref, y)


def _k2_body(y1_ref, cs_ref, w_ref, y2_ref, st_ref):
    sc = cs_ref[0:1, :]
    sh = cs_ref[1:2, :]
    a = jax.nn.relu(y1_ref[...] * sc + sh)
    y2 = jnp.dot(a, w_ref[...], preferred_element_type=jnp.float32)
    y2_ref[...] = y2
    _acc_stats(st_ref, y2)


def _k3_body(y2_ref, x_ref, cs_ref, w_ref, v1_ref, y3_ref, st_ref):
    sc = cs_ref[0:1, :]
    sh = cs_ref[1:2, :]
    v1 = jax.nn.relu(y2_ref[...] * sc + sh + x_ref[...])
    v1_ref[...] = v1
    y3 = jnp.dot(v1, w_ref[...], preferred_element_type=jnp.float32)
    y3_ref[...] = y3
    _acc_stats(st_ref, y3)


def _k5_body(y4_ref, v1_ref, x_ref, cs_ref, wp1_ref, bp1_ref,
             v_ref, vsc_ref, g1_ref):
    sc = cs_ref[0:1, :]
    sh = cs_ref[1:2, :]
    vv = jax.nn.relu(y4_ref[...] * sc + sh + v1_ref[...])
    v_ref[...] = vv
    vsc_ref[0] = vv[:, :H]
    vsc_ref[1] = vv[:, H:]
    f = x_ref[...] + vv
    g1_ref[...] = (jnp.dot(f, wp1_ref[...], preferred_element_type=jnp.float32)
                   + bp1_ref[0:1, :])


def _t3a_body(ds_ref, ic_ref, st_ref):
    h1 = _leaky(ds_ref[...] * ic_ref[...])
    _acc_stats(st_ref, h1)


def _t3b_body(ds_ref, ic_ref, cs1_ref, w_ref, b_ref, st_ref):
    h1 = _leaky(ds_ref[...] * ic_ref[...])
    h1n = h1 * cs1_ref[0:1, :] + cs1_ref[1:2, :]
    h2p = _leaky(jnp.dot(h1n, w_ref[...], preferred_element_type=jnp.float32)
                 + b_ref[0:1, :])
    _acc_stats(st_ref, h2p)


def _t3c_body(ds_ref, ic_ref, cs1_ref, w_ref, b_ref, cs2_ref, h2_ref):
    h1 = _leaky(ds_ref[...] * ic_ref[...])
    h1n = h1 * cs1_ref[0:1, :] + cs1_ref[1:2, :]
    h2p = _leaky(jnp.dot(h1n, w_ref[...], preferred_element_type=jnp.float32)
                 + b_ref[0:1, :])
    h2_ref[...] = h2p * cs2_ref[0:1, :] + cs2_ref[1:2, :]


def _tc4_body(x_ref, vsc_ref, h2g_ref, wli_ref, wp3_ref, wo1a_ref, wo1b_ref,
              wo2_ref, bv_ref, losc_ref):
    v = jnp.concatenate([vsc_ref[0], vsc_ref[1]], axis=1)
    f = x_ref[...] + v
    idt = _leaky(jnp.dot(f, wli_ref[...], preferred_element_type=jnp.float32)
                 + bv_ref[0:1, :])
    h3 = _leaky(jnp.dot(h2g_ref[...], wp3_ref[...],
                        preferred_element_type=jnp.float32) + bv_ref[1:2, :])
    t = _leaky(jnp.dot(idt, wo1a_ref[...], preferred_element_type=jnp.float32)
               + jnp.dot(h3, wo1b_ref[...], preferred_element_type=jnp.float32)
               + bv_ref[2:3, :])
    lo = (jnp.dot(t, wo2_ref[...], preferred_element_type=jnp.float32)
          + bv_ref[3:4, :])
    losc_ref[0] = lo[:, :H]
    losc_ref[1] = lo[:, H:]


def _tc5_body(sv_ref, sl_ref, ic_ref, nf_ref, pf_ref):
    ic = ic_ref[...]
    pf = jnp.concatenate([sl_ref[0], sl_ref[1]], axis=1) * ic
    vf = jnp.concatenate([sv_ref[0], sv_ref[1]], axis=1) * ic
    pf_ref[...] = pf
    nf_ref[...] = pf + vf


def _bspec(shape, idx):
    return pl.BlockSpec(shape, idx)


_FULLW = pl.BlockSpec((C, C), lambda i: (0, 0))
_FULLH = pl.BlockSpec((H, H), lambda i: (0, 0))
_ROWB = pl.BlockSpec((RB, C), lambda i: (i, 0))
_ROWBH = pl.BlockSpec((RB, H), lambda i: (i, 0))
_CS = pl.BlockSpec((8, C), lambda i: (0, 0))
_CSH = pl.BlockSpec((8, H), lambda i: (0, 0))
_ST = pl.BlockSpec((8, C), lambda i: (0, 0))
_STH = pl.BlockSpec((8, H), lambda i: (0, 0))
_SPLITB = pl.BlockSpec((2, RB, H), lambda i: (0, i, 0))
_IC = pl.BlockSpec((RB, 1), lambda i: (i, 0))


def _call(body, in_specs, out_specs, out_shapes, grid=GRID):
    return pl.pallas_call(
        body,
        grid=(grid,),
        in_specs=in_specs,
        out_specs=out_specs,
        out_shape=out_shapes,
    )


def _bn_coeffs(st, n, g, b):
    s, ss = st[0], st[1]
    m = s / n
    var = ss / n - m * m
    sc = g / jnp.sqrt(var + 1e-5)
    sh = b - m * sc
    return jnp.stack([sc, sh] + [jnp.zeros_like(sc)] * 6, axis=0)


def _share(s, n, parts):
    base = n // parts
    rem = n % parts
    cnt = base + jnp.where(s < rem, 1, 0)
    start = s * base + jnp.minimum(s, rem)
    return start, cnt


def _zero_vmem(ref, rows, width):
    for r in range(rows):
        for j in range(width // 16):
            ref[r, pl.ds(j * 16, 16)] = jnp.zeros((16,), jnp.float32)


def _fill_ones(ref, rows, width):
    for r in range(rows):
        for j in range(width // 16):
            ref[r, pl.ds(j * 16, 16)] = jnp.ones((16,), jnp.float32)


def _sc_mesh():
    return plsc.VectorSubcoreMesh(core_axis_name="c", subcore_axis_name="s",
                                  num_cores=NC, num_subcores=NS)


def _zero_spmem_rows(zbuf, shref, n_rows, s, zr=ZR):
    nch = n_rows // zr
    z0, zc = _share(s, nch, NS)

    def zbody(i, _):
        pltpu.sync_copy(zbuf, shref.at[pl.ds(i * zr, zr)])
        return 0

    lax.fori_loop(z0, z0 + zc, zbody, 0)
    rem = n_rows - nch * zr
    if rem:
        @pl.when(s == 0)
        def _():
            pltpu.sync_copy(zbuf.at[pl.ds(0, rem)],
                            shref.at[pl.ds(nch * zr, rem)])


def _copy_spmem_rows(shref, dst, dst_base, n_rows, s):
    nch = n_rows // ZR
    w0, wc = _share(s, nch, NS)

    def wbody(i, _):
        pltpu.sync_copy(shref.at[pl.ds(i * ZR, ZR)],
                        dst.at[pl.ds(dst_base + i * ZR, ZR)])
        return 0

    lax.fori_loop(w0, w0 + wc, wbody, 0)
    rem = n_rows - nch * ZR
    if rem:
        @pl.when(s == 0)
        def _():
            pltpu.sync_copy(shref.at[pl.ds(nch * ZR, rem)],
                            dst.at[pl.ds(dst_base + nch * ZR, rem)])


def _make_scatter_stage():
    out_type = [jax.ShapeDtypeStruct((2 * N_DOWN, H), jnp.float32)]

    def body(table, last1, scale1, sums,
             il0, il1, is0, is1, r0, r1,
             sil0, sil1, sis0, sis1, sg0, sg1, acc):
        c = lax.axis_index("c")
        s = lax.axis_index("s")
        off = c * N_V
        ils = [il0, il1]
        iss = [is0, is1]
        rs = [r0, r1]
        sils = [sil0, sil1]
        siss = [sis0, sis1]
        sgs = [sg0, sg1]

        _zero_vmem(r0, PROW, H)
        _zero_spmem_rows(r0, acc, N_DOWN, s, PROW)
        plsc.subcore_barrier()

        g0, gc = _share(s, NPR, NS)

        def start_idx(g, b):
            pltpu.async_copy(last1.at[pl.ds((g0 + g) * PROW, PROW)],
                             ils[b], sils[b])
            pltpu.async_copy(scale1.at[pl.ds((g0 + g) * PROW, PROW)],
                             iss[b], siss[b])

        def wait_idx(g, b):
            pltpu.make_async_copy(last1.at[pl.ds((g0 + g) * PROW, PROW)],
                                  ils[b], sils[b]).wait()
            pltpu.make_async_copy(scale1.at[pl.ds((g0 + g) * PROW, PROW)],
                                  iss[b], siss[b]).wait()

        def wait_gather_scatter(b):
            pltpu.make_async_copy(table.at[ils[b]], rs[b], sgs[b]).wait()
            pltpu.sync_copy(rs[b], acc.at[iss[b]], add=True)

        @pl.when(gc > 0)
        def _():
            start_idx(0, 0)

        def outer(o, _):
            for b in range(2):
                g = o * 2 + b

                @pl.when(g < gc)
                def _(g=g, b=b):
                    wait_idx(g, b)
                    for j in range(PROW // 16):
                        ils[b][pl.ds(j * 16, 16)] = (
                            ils[b][pl.ds(j * 16, 16)] + off)
                    pltpu.async_copy(table.at[ils[b]], rs[b], sgs[b])

                    @pl.when(g > 0)
                    def _():
                        wait_gather_scatter(1 - b)

                    @pl.when(g + 1 < gc)
                    def _():
                        start_idx(g + 1, 1 - b)
            return 0

        lax.fori_loop(0, (gc + 1) // 2, outer, 0)
        for b in range(2):
            @pl.when((gc > 0) & (lax.rem(gc - 1, 2) == b))
            def _(b=b):
                wait_gather_scatter(b)

        plsc.subcore_barrier()
        _copy_spmem_rows(acc, sums, c * N_DOWN, N_DOWN, s)

    scratch = [
        pltpu.VMEM((PROW,), jnp.int32),
        pltpu.VMEM((PROW,), jnp.int32),
        pltpu.VMEM((PROW,), jnp.int32),
        pltpu.VMEM((PROW,), jnp.int32),
        pltpu.VMEM((PROW, H), jnp.float32),
        pltpu.VMEM((PROW, H), jnp.float32),
        pltpu.SemaphoreType.DMA,
        pltpu.SemaphoreType.DMA,
        pltpu.SemaphoreType.DMA,
        pltpu.SemaphoreType.DMA,
        pltpu.SemaphoreType.DMA,
        pltpu.SemaphoreType.DMA,
        pltpu.VMEM_SHARED((N_DOWN, H), jnp.float32),
    ]
    return pl.kernel(body, out_type=out_type, mesh=_sc_mesh(),
                     scratch_types=scratch,
                     compiler_params=pltpu.CompilerParams(
                         use_tc_tiling_on_sc=False))


def _make_counts_stage():
    HALF = N_V // 2
    out_type = [jax.ShapeDtypeStruct((N_V, 16), jnp.float32),
                jax.ShapeDtypeStruct((N_DOWN, 16), jnp.float32)]

    def body(inv1, scale1, cnt100, cnt25, i80a, i80b, ia, ib, s0, s1,
             ones80, ones, zbuf, acc):
        c = lax.axis_index("c")
        s = lax.axis_index("s")
        _fill_ones(ones, PROW, 16)
        _fill_ones(ones80, VROW, 16)
        _zero_vmem(zbuf, ZR, 16)
        sms = [s0, s1]

        @pl.when(c == 0)
        def _():
            idxs = [i80a, i80b]
            for kk in range(2):
                base = kk * HALF
                _zero_spmem_rows(zbuf, acc, HALF + 8, s)
                plsc.subcore_barrier()
                g0, gc = _share(s, NVR, NS)

                def start_ld(g, b):
                    pltpu.async_copy(inv1.at[pl.ds((g0 + g) * VROW, VROW)],
                                     idxs[b], sms[b])

                def wait_ld(g, b):
                    pltpu.make_async_copy(
                        inv1.at[pl.ds((g0 + g) * VROW, VROW)],
                        idxs[b], sms[b]).wait()

                @pl.when(gc > 0)
                def _():
                    start_ld(0, 0)

                def outer(o, _):
                    for b in range(2):
                        g = o * 2 + b

                        @pl.when(g < gc)
                        def _(g=g, b=b):
                            wait_ld(g, b)

                            @pl.when(g + 1 < gc)
                            def _():
                                start_ld(g + 1, 1 - b)

                            for j in range(VROW // 16):
                                t = idxs[b][pl.ds(j * 16, 16)] - base
                                ok = (t >= 0) & (t < HALF)
                                idxs[b][pl.ds(j * 16, 16)] = jnp.where(
                                    ok, t, HALF)
                            pltpu.sync_copy(ones80, acc.at[idxs[b]],
                                            add=True)
                    return 0

                lax.fori_loop(0, (gc + 1) // 2, outer, 0)
                plsc.subcore_barrier()
                _copy_spmem_rows(acc, cnt100, base, HALF, s)
                plsc.subcore_barrier()

        @pl.when(c == 1)
        def _():
            idxs = [ia, ib]
            _zero_spmem_rows(zbuf, acc, N_DOWN, s)
            plsc.subcore_barrier()
            g0, gc = _share(s, NPR, NS)

            def start_ld(g, b):
                pltpu.async_copy(scale1.at[pl.ds((g0 + g) * PROW, PROW)],
                                 idxs[b], sms[b])

            def wait_ld(g, b):
                pltpu.make_async_copy(scale1.at[pl.ds((g0 + g) * PROW, PROW)],
                                      idxs[b], sms[b]).wait()

            @pl.when(gc > 0)
            def _():
                start_ld(0, 0)

            def outer(o, _):
                for b in range(2):
                    g = o * 2 + b

                    @pl.when(g < gc)
                    def _(g=g, b=b):
                        wait_ld(g, b)

                        @pl.when(g + 1 < gc)
                        def _():
                            start_ld(g + 1, 1 - b)

                        pltpu.sync_copy(ones, acc.at[idxs[b]], add=True)
                return 0

            lax.fori_loop(0, (gc + 1) // 2, outer, 0)
            plsc.subcore_barrier()
            _copy_spmem_rows(acc, cnt25, 0, N_DOWN, s)

    scratch = [
        pltpu.VMEM((VROW,), jnp.int32),
        pltpu.VMEM((VROW,), jnp.int32),
        pltpu.VMEM((PROW,), jnp.int32),
        pltpu.VMEM((PROW,), jnp.int32),
        pltpu.SemaphoreType.DMA,
        pltpu.SemaphoreType.DMA,
        pltpu.VMEM((VROW, 16), jnp.float32),
        pltpu.VMEM((PROW, 16), jnp.float32),
        pltpu.VMEM((ZR, 16), jnp.float32),
        pltpu.VMEM_SHARED((HALF + 8, 16), jnp.float32),
    ]
    return pl.kernel(body, out_type=out_type, mesh=_sc_mesh(),
                     scratch_types=scratch,
                     compiler_params=pltpu.CompilerParams(
                         use_tc_tiling_on_sc=False))


def _make_down_stage():
    out_type = [jax.ShapeDtypeStruct((N_V, H), jnp.float32)]

    def body(g1, inv1, dsum, i0, i1, i2, v0, v1, v2,
             si0, si1, si2, sv0, sv1, sv2, ss0, ss1, ss2, zbuf, acc):
        c = lax.axis_index("c")
        s = lax.axis_index("s")
        _zero_vmem(zbuf, ZR, H)
        idxs = [i0, i1, i2]
        vs = [v0, v1, v2]
        sis = [si0, si1, si2]
        svs = [sv0, sv1, sv2]
        sss = [ss0, ss1, ss2]

        for kk in range(2):
            base = (2 * c + kk) * N_DOWN
            _zero_spmem_rows(zbuf, acc, N_DOWN + 8, s)
            plsc.subcore_barrier()

            g0, gc = _share(s, NVR, NS)

            def start_ld(g, b):
                pltpu.async_copy(inv1.at[pl.ds((g0 + g) * VROW, VROW)],
                                 idxs[b], sis[b])
                pltpu.async_copy(g1.at[pl.ds((g0 + g) * VROW, VROW)],
                                 vs[b], svs[b])

            def wait_ld(g, b):
                pltpu.make_async_copy(inv1.at[pl.ds((g0 + g) * VROW, VROW)],
                                      idxs[b], sis[b]).wait()
                pltpu.make_async_copy(g1.at[pl.ds((g0 + g) * VROW, VROW)],
                                      vs[b], svs[b]).wait()

            def wait_scatter(b):
                pltpu.make_async_copy(vs[b], acc.at[idxs[b]], sss[b]).wait()

            @pl.when(gc > 0)
            def _():
                start_ld(0, 0)

            def outer(o, _):
                for b in range(3):
                    g = o * 3 + b

                    @pl.when(g < gc)
                    def _(g=g, b=b):
                        wait_ld(g, b)
                        for j in range(VROW // 16):
                            t = idxs[b][pl.ds(j * 16, 16)] - base
                            ok = (t >= 0) & (t < N_DOWN)
                            idxs[b][pl.ds(j * 16, 16)] = jnp.where(
                                ok, t, N_DOWN)
                        pltpu.async_copy(vs[b], acc.at[idxs[b]], sss[b],
                                         add=True)

                        @pl.when(g > 1)
                        def _():
                            wait_scatter((b + 1) % 3)

                        @pl.when(g + 1 < gc)
                        def _():
                            start_ld(g + 1, (b + 1) % 3)
                return 0

            lax.fori_loop(0, (gc + 2) // 3, outer, 0)
            for b in range(3):
                @pl.when((gc > 1) & (lax.rem(gc - 2, 3) == b))
                def _(b=b):
                    wait_scatter(b)

                @pl.when((gc > 0) & (lax.rem(gc - 1, 3) == b))
                def _(b=b):
                    wait_scatter(b)

            plsc.subcore_barrier()

            _copy_spmem_rows(acc, dsum, base, N_DOWN, s)
            plsc.subcore_barrier()

    scratch = (
        [pltpu.VMEM((VROW,), jnp.int32) for _ in range(3)]
        + [pltpu.VMEM((VROW, H), jnp.float32) for _ in range(3)]
        + [pltpu.SemaphoreType.DMA for _ in range(9)]
        + [pltpu.VMEM((ZR, H), jnp.float32),
           pltpu.VMEM_SHARED((N_DOWN + 8, H), jnp.float32)]
    )
    return pl.kernel(body, out_type=out_type, mesh=_sc_mesh(),
                     scratch_types=scratch,
                     compiler_params=pltpu.CompilerParams(
                         use_tc_tiling_on_sc=False))


def _make_gather_stage(width, n_chunks, per_row):
    out_type = [jax.ShapeDtypeStruct((n_chunks * per_row, width),
                                     jnp.float32)]

    def body(tab, idx1, out, i0, i1, b0, b1, si0, si1, sg0, sg1, sw0, sw1):
        c = lax.axis_index("c")
        s = lax.axis_index("s")
        wid = s * NC + c
        g0, gc = _share(wid, n_chunks, NC * NS)
        idxs = [i0, i1]
        bufs = [b0, b1]
        sis = [si0, si1]
        sgs = [sg0, sg1]
        sws = [sw0, sw1]

        def start_idx(g, b):
            pltpu.async_copy(idx1.at[pl.ds((g0 + g) * per_row, per_row)],
                             idxs[b], sis[b])

        def wait_idx(g, b):
            pltpu.make_async_copy(idx1.at[pl.ds((g0 + g) * per_row, per_row)],
                                  idxs[b], sis[b]).wait()

        def start_write(g, b):
            pltpu.async_copy(bufs[b],
                             out.at[pl.ds((g0 + g) * per_row, per_row)],
                             sws[b])

        def wait_write(g, b):
            pltpu.make_async_copy(
                bufs[b], out.at[pl.ds((g0 + g) * per_row, per_row)],
                sws[b]).wait()

        @pl.when(gc > 0)
        def _():
            start_idx(0, 0)

        def outer(o, _):
            for b in range(2):
                g = o * 2 + b

                @pl.when(g < gc)
                def _(g=g, b=b):
                    wait_idx(g, b)

                    @pl.when(g > 1)
                    def _():
                        wait_write(g - 2, b)

                    pltpu.async_copy(tab.at[idxs[b]], bufs[b], sgs[b])

                    @pl.when(g > 0)
                    def _():
                        pltpu.make_async_copy(tab.at[idxs[1 - b]],
                                              bufs[1 - b], sgs[1 - b]).wait()
                        start_write(g - 1, 1 - b)

                    @pl.when(g + 1 < gc)
                    def _():
                        start_idx(g + 1, 1 - b)
            return 0

        lax.fori_loop(0, (gc + 1) // 2, outer, 0)
        for b in range(2):
            @pl.when((gc > 1) & (lax.rem(gc - 2, 2) == b))
            def _(b=b):
                wait_write(gc - 2, b)

            @pl.when((gc > 0) & (lax.rem(gc - 1, 2) == b))
            def _(b=b):
                pltpu.make_async_copy(tab.at[idxs[b]], bufs[b],
                                      sgs[b]).wait()
                pltpu.sync_copy(bufs[b],
                                out.at[pl.ds((g0 + gc - 1) * per_row,
                                             per_row)])

    scratch = [
        pltpu.VMEM((per_row,), jnp.int32),
        pltpu.VMEM((per_row,), jnp.int32),
        pltpu.VMEM((per_row, width), jnp.float32),
        pltpu.VMEM((per_row, width), jnp.float32),
        pltpu.SemaphoreType.DMA,
        pltpu.SemaphoreType.DMA,
        pltpu.SemaphoreType.DMA,
        pltpu.SemaphoreType.DMA,
        pltpu.SemaphoreType.DMA,
        pltpu.SemaphoreType.DMA,
    ]
    return pl.kernel(body, out_type=out_type, mesh=_sc_mesh(),
                     scratch_types=scratch,
                     compiler_params=pltpu.CompilerParams(
                         use_tc_tiling_on_sc=False))


def kernel(features, coors, coors_inv_last, coors_inv_scale, params):
    p = params
    nvf = jnp.float32(N_V)

    y1, st1 = _call(_k1_body, [_ROWB, _FULLW], [_ROWB, _ST],
                    [jax.ShapeDtypeStruct((N_V, C), jnp.float32),
                     jax.ShapeDtypeStruct((8, C), jnp.float32)])(
                         features, p['W1a'])
    cs1 = _bn_coeffs(st1, nvf, p['g1a'], p['b1a'])
    y2, st2 = _call(_k2_body, [_ROWB, _CS, _FULLW], [_ROWB, _ST],
                    [jax.ShapeDtypeStruct((N_V, C), jnp.float32),
                     jax.ShapeDtypeStruct((8, C), jnp.float32)])(
                         y1, cs1, p['W1b'])
    cs2 = _bn_coeffs(st2, nvf, p['g1b'], p['b1b'])
    v1, y3, st3 = _call(_k3_body, [_ROWB, _ROWB, _CS, _FULLW],
                        [_ROWB, _ROWB, _ST],
                        [jax.ShapeDtypeStruct((N_V, C), jnp.float32),
                         jax.ShapeDtypeStruct((N_V, C), jnp.float32),
                         jax.ShapeDtypeStruct((8, C), jnp.float32)])(
                            y2, features, cs2, p['W2a'])
    cs3 = _bn_coeffs(st3, nvf, p['g2a'], p['b2a'])
    y4, st4 = _call(_k2_body, [_ROWB, _CS, _FULLW], [_ROWB, _ST],
                    [jax.ShapeDtypeStruct((N_V, C), jnp.float32),
                     jax.ShapeDtypeStruct((8, C), jnp.float32)])(
                         y3, cs3, p['W2b'])
    cs4 = _bn_coeffs(st4, nvf, p['g2b'], p['b2b'])
    bp1v = jnp.stack([p['bp1']] + [jnp.zeros((H,), jnp.float32)] * 7, axis=0)
    wp1spec = pl.BlockSpec((C, H), lambda i: (0, 0))
    v, vsc, g1 = _call(_k5_body,
                       [_ROWB, _ROWB, _ROWB, _CS, wp1spec, _CSH],
                       [_ROWB, _SPLITB, _ROWBH],
                       [jax.ShapeDtypeStruct((N_V, C), jnp.float32),
                        jax.ShapeDtypeStruct((2, N_V, H), jnp.float32),
                        jax.ShapeDtypeStruct((N_V, H), jnp.float32)])(
                           y4, v1, features, cs4, p['Wp1'], bp1v)

    co = coors.astype(jnp.int32)
    b = co[:, 0]
    xyz = co[:, 1:] // 2
    keyv = ((b * 64 + xyz[:, 0]) * 64 + xyz[:, 1]) * 64 + xyz[:, 2]
    order = jnp.argsort(keyv)
    sk = keyv[order]
    flag = jnp.concatenate(
        [jnp.zeros((1,), sk.dtype), (sk[1:] != sk[:-1]).astype(sk.dtype)])
    gid = jnp.cumsum(flag)
    inv = jnp.zeros((N_V,), jnp.int32).at[order].set(gid.astype(jnp.int32))
    n_u = (gid[-1] + 1).astype(jnp.int32)

    vtab = vsc.reshape(2 * N_V, H)

    sumv, = _make_scatter_stage()(vtab, coors_inv_last, coors_inv_scale)
    cnt100v, cnt25v = _make_counts_stage()(inv, coors_inv_scale)
    invc25 = (1.0 / jnp.clip(cnt25v[:, 0], 1.0)).reshape(N_DOWN, 1)

    dsum, = _make_down_stage()(g1, inv)
    invc100 = (1.0 / jnp.clip(cnt100v[:, 0], 1.0)).reshape(N_V, 1)

    nf = n_u.astype(jnp.float32)
    st_h1, = _call(_t3a_body, [_ROWBH, _IC], [_STH],
                   [jax.ShapeDtypeStruct((8, H), jnp.float32)])(dsum, invc100)
    csA = _bn_coeffs(st_h1, nf, p['g1'], p['be1'])
    bp2v = jnp.stack([p['bp2']] + [jnp.zeros((H,), jnp.float32)] * 7, axis=0)
    st_h2, = _call(_t3b_body, [_ROWBH, _IC, _CSH, _FULLH, _CSH], [_STH],
                   [jax.ShapeDtypeStruct((8, H), jnp.float32)])(
                       dsum, invc100, csA, p['Wp2'], bp2v)
    c2 = _leaky(csA[1] @ p['Wp2'] + p['bp2'])
    ng = nvf - nf
    s2m = st_h2[0] - ng * c2
    ss2m = st_h2[1] - ng * c2 * c2
    csB = _bn_coeffs(jnp.stack([s2m, ss2m] + [jnp.zeros((H,))] * 6), nf,
                     p['g2'], p['be2'])
    h2, = _call(_t3c_body, [_ROWBH, _IC, _CSH, _FULLH, _CSH, _CSH], [_ROWBH],
                [jax.ShapeDtypeStruct((N_V, H), jnp.float32)])(
                    dsum, invc100, csA, p['Wp2'], bp2v, csB)

    h2g, = _make_gather_stage(H, NVR, VROW)(h2, inv)

    bv = jnp.stack([p['bli'], p['bp3'], p['bo1'], p['bo2']]
                   + [jnp.zeros((C,), jnp.float32)] * 4, axis=0)
    wp3spec = pl.BlockSpec((H, C), lambda i: (0, 0))
    losc, = _call(_tc4_body,
                  [_ROWB, _SPLITB, _ROWBH, _FULLW, wp3spec, _FULLW, _FULLW,
                   _FULLW, _CS],
                  [_SPLITB],
                  [jax.ShapeDtypeStruct((2, N_V, H), jnp.float32)])(
                      features, vsc, h2g, p['Wli'], p['Wp3'],
                      p['Wo1'][:C], p['Wo1'][C:], p['Wo2'], bv)

    sumlo, = _make_scatter_stage()(losc.reshape(2 * N_V, H),
                                   coors_inv_last, coors_inv_scale)

    svr = sumv.reshape(2, N_DOWN, H)
    slr = sumlo.reshape(2, N_DOWN, H)
    spl25 = pl.BlockSpec((2, RB25, H), lambda i: (0, i, 0))
    ic25 = pl.BlockSpec((RB25, 1), lambda i: (i, 0))
    row25 = pl.BlockSpec((RB25, C), lambda i: (i, 0))
    new_feats, p_fea = _call(
        _tc5_body, [spl25, spl25, ic25], [row25, row25],
        [jax.ShapeDtypeStruct((N_DOWN, C), jnp.float32),
         jax.ShapeDtypeStruct((N_DOWN, C), jnp.float32)],
        grid=N_DOWN // RB25)(svr, slr, invc25)

    pts_feat_f, = _make_gather_stage(C, NPR, PROW)(p_fea, coors_inv_scale)

    return (new_feats, v, pts_feat_f)
